# baseline (speedup 1.0000x reference)
import functools

import jax
import jax.numpy as jnp
from jax import lax
from jax.experimental import pallas as pl
from jax.experimental.pallas import tpu as pltpu
from jax.experimental.pallas import tpu_sc as plsc

N = 10000
NPAD = 10240
F = 128
FH = F // 2
LANES = 16
NC = 2
NS = 16
CHUNK = 128
NBUF = 5


def _cheb_sc(nchunks):
    rpt = NPAD // NS
    vpc = CHUNK // LANES
    assert nchunks % NBUF == 0
    nrounds = nchunks // NBUF

    mesh = plsc.VectorSubcoreMesh(
        core_axis_name="c", subcore_axis_name="s", num_cores=NC, num_subcores=NS
    )

    @functools.partial(
        pl.kernel,
        out_type=(
            jax.ShapeDtypeStruct((NC, NPAD, FH), jnp.float32),
            jax.ShapeDtypeStruct((NC, NPAD, FH), jnp.float32),
        ),
        mesh=mesh,
        compiler_params=pltpu.CompilerParams(use_tc_tiling_on_sc=False),
        scratch_types=[
            pltpu.VMEM((NBUF, CHUNK), jnp.int32),
            pltpu.VMEM((NBUF, CHUNK), jnp.int32),
            pltpu.VMEM((NBUF * vpc, LANES), jnp.float32),
            pltpu.VMEM((NBUF, CHUNK, FH), jnp.float32),
            pltpu.VMEM_SHARED((NPAD, FH), jnp.float32),
            pltpu.VMEM_SHARED((NPAD, FH), jnp.float32),
            [pltpu.SemaphoreType.DMA] * NBUF,
            [pltpu.SemaphoreType.DMA] * NBUF,
            [pltpu.SemaphoreType.DMA] * NBUF,
            pltpu.SemaphoreType.DMA,
            pltpu.SemaphoreType.DMA,
        ],
    )
    def cheb(y_hbm, cols_hbm, rows_hbm, vals_hbm, t1_hbm, s2_hbm,
             cidx, ridx, vvals, g, spa, spb, semi, semg, sems, semwb, semz):
        c = lax.axis_index("c")
        s = lax.axis_index("s")
        base = s * rpt

        def issue_idx(b, chunk):
            pltpu.async_copy(cols_hbm.at[s, chunk], cidx.at[b], semi[b])
            pltpu.async_copy(rows_hbm.at[s, chunk], ridx.at[b], semi[b])
            pltpu.async_copy(
                vals_hbm.at[s, pl.ds(chunk * vpc, vpc)],
                vvals.at[pl.ds(b * vpc, vpc)], semi[b],
            )

        def wait_idx(b):
            pltpu.make_async_copy(cols_hbm.at[s, 0], cidx.at[b], semi[b]).wait()
            pltpu.make_async_copy(rows_hbm.at[s, 0], ridx.at[b], semi[b]).wait()
            pltpu.make_async_copy(
                vals_hbm.at[s, pl.ds(0, vpc)],
                vvals.at[pl.ds(b * vpc, vpc)], semi[b],
            ).wait()

        def zero_acc(dst):
            def zero_g(i, carry):
                for f in range(FH // LANES):
                    g[0, i, pl.ds(f * LANES, LANES)] = jnp.zeros(
                        (LANES,), jnp.float32
                    )
                return carry

            lax.fori_loop(0, CHUNK, zero_g, 0)
            for off in range(0, rpt, CHUNK):
                pltpu.async_copy(g.at[0], dst.at[pl.ds(base + off, CHUNK)], semz)

        def drain_zero(dst):
            for off in range(0, rpt, CHUNK):
                pltpu.make_async_copy(
                    g.at[0], dst.at[pl.ds(base + off, CHUNK)], semz
                ).wait()

        def run_pass(src, dst):
            def issue_gather(b):
                pltpu.async_copy(src.at[cidx.at[b]], g.at[b], semg[b])

            def wait_gather(b):
                pltpu.make_async_copy(
                    src.at[pl.ds(0, CHUNK)], g.at[b], semg[b]
                ).wait()

            def issue_scatter(b):
                pltpu.async_copy(g.at[b], dst.at[ridx.at[b]], sems[b], add=True)

            def wait_scatter(b):
                pltpu.make_async_copy(
                    g.at[b], dst.at[pl.ds(0, CHUNK)], sems[b]
                ).wait()

            for b in range(3):
                wait_idx(b)
                issue_gather(b)

            def round_body(k, carry):
                for b in range(NBUF):
                    wait_gather(b)

                    @plsc.parallel_loop(0, vpc, 1, unroll=8)
                    def scale_group(t):
                        vv16 = vvals[b * vpc + t, :]
                        for l in range(LANES):
                            vv = jnp.full((LANES,), vv16[l], jnp.float32)
                            row = t * LANES + l
                            for f in range(FH // LANES):
                                sl = pl.ds(f * LANES, LANES)
                                g[b, row, sl] = g[b, row, sl] * vv

                    issue_scatter(b)

                    bA = (b + 3) % NBUF

                    def _refill_a():
                        wait_scatter(bA)
                        issue_idx(bA, k * NBUF + b + 3)

                    if b < 2:
                        @pl.when(k > 0)
                        def _():
                            _refill_a()
                    else:
                        @pl.when(k < nrounds - 1)
                        def _():
                            _refill_a()

                    bB = (b + 2) % NBUF

                    def _refill_b():
                        wait_idx(bB)
                        issue_gather(bB)

                    if b == 0:
                        @pl.when(k > 0)
                        def _():
                            _refill_b()
                    elif b < 3:
                        _refill_b()
                    else:
                        @pl.when(k < nrounds - 1)
                        def _():
                            _refill_b()
                return carry

            lax.fori_loop(0, nrounds, round_body, 0)
            for b in range(NBUF):
                wait_scatter(b)

        for b in range(NBUF):
            issue_idx(b, b)
        zero_acc(spb)

        n_y = y_hbm.shape[0]
        for cc in range(NC):
            stage_rows = min(rpt, n_y - (NS - 1) * rpt)

            @pl.when((c == cc) & (s < NS - 1))
            def _():
                pltpu.sync_copy(
                    y_hbm.at[pl.ds(base, rpt), pl.ds(cc * FH, FH)],
                    spa.at[pl.ds(base, rpt)],
                )

            @pl.when((c == cc) & (s == NS - 1))
            def _():
                pltpu.sync_copy(
                    y_hbm.at[pl.ds((NS - 1) * rpt, stage_rows), pl.ds(cc * FH, FH)],
                    spa.at[pl.ds((NS - 1) * rpt, stage_rows)],
                )

        drain_zero(spb)
        plsc.subcore_barrier()

        run_pass(spa, spb)
        plsc.subcore_barrier()

        wb = pltpu.async_copy(
            spb.at[pl.ds(base, rpt)], t1_hbm.at[c, pl.ds(base, rpt)], semwb
        )
        for b in range(NBUF):
            issue_idx(b, b)
        zero_acc(spa)
        drain_zero(spa)
        plsc.subcore_barrier()

        run_pass(spb, spa)
        plsc.subcore_barrier()
        pltpu.sync_copy(spa.at[pl.ds(base, rpt)], s2_hbm.at[c, pl.ds(base, rpt)])
        wb.wait()

    return cheb


_BR = 2000


def _finalize(x, t1, s2, theta, bias2d):
    def body(x_ref, t1_ref, s2_ref, th_ref, bias_ref, o_ref):
        wa = th_ref[0] - th_ref[2]
        wb = th_ref[1]
        wc = 2.0 * th_ref[2]
        o = (
            jnp.dot(x_ref[...], wa, preferred_element_type=jnp.float32)
            + jnp.dot(t1_ref[0], wb[:FH], preferred_element_type=jnp.float32)
            + jnp.dot(t1_ref[1], wb[FH:], preferred_element_type=jnp.float32)
            + jnp.dot(s2_ref[0], wc[:FH], preferred_element_type=jnp.float32)
            + jnp.dot(s2_ref[1], wc[FH:], preferred_element_type=jnp.float32)
        )
        o_ref[...] = jnp.maximum(o + bias_ref[...], 0.0)

    return pl.pallas_call(
        body,
        grid=(N // _BR,),
        in_specs=[
            pl.BlockSpec((_BR, F), lambda i: (i, 0)),
            pl.BlockSpec((NC, _BR, FH), lambda i: (0, i, 0)),
            pl.BlockSpec((NC, _BR, FH), lambda i: (0, i, 0)),
            pl.BlockSpec((3, F, F), lambda i: (0, 0, 0)),
            pl.BlockSpec((1, F), lambda i: (0, 0)),
        ],
        out_specs=pl.BlockSpec((_BR, F), lambda i: (i, 0)),
        out_shape=jax.ShapeDtypeStruct((N, F), jnp.float32),
    )(x, t1, s2, theta, bias2d)


def kernel(x, rows, cols, vals, theta, bias):
    x = x.astype(jnp.float32)
    e = rows.shape[0]
    per_super = NS * CHUNK
    nchunks = -(-e // per_super)
    nchunks = -(-nchunks // NBUF) * NBUF
    e_pad = nchunks * per_super
    pad = e_pad - e
    if pad:
        spread = (jnp.arange(pad, dtype=jnp.int32) * 37) % N
        rows = jnp.concatenate([rows, spread])
        cols = jnp.concatenate([cols, spread])
        vals = jnp.pad(vals, (0, pad))
    cols2 = cols.reshape(NS, nchunks, 128)
    rows2 = rows.reshape(NS, nchunks, 128)
    vals2 = vals.reshape(NS, nchunks * (CHUNK // LANES), LANES)

    t1, s2 = _cheb_sc(nchunks)(x, cols2, rows2, vals2)
    return _finalize(x, t1, s2, theta, bias.reshape(1, F))


# speedup vs baseline: 1.1862x; 1.1862x over previous
import functools

import jax
import jax.numpy as jnp
from jax import lax
from jax.experimental import pallas as pl
from jax.experimental.pallas import tpu as pltpu
from jax.experimental.pallas import tpu_sc as plsc

N = 10000
NPAD = 10240
F = 128
FH = F // 2
LANES = 16
NC = 2
NS = 16
CHUNK = 128
NBUF = 5


def _cheb_sc(nchunks):
    rpt = NPAD // NS
    vpc = CHUNK // LANES
    assert nchunks % NBUF == 0
    nrounds = nchunks // NBUF

    mesh = plsc.VectorSubcoreMesh(
        core_axis_name="c", subcore_axis_name="s", num_cores=NC, num_subcores=NS
    )

    @functools.partial(
        pl.kernel,
        out_type=(
            jax.ShapeDtypeStruct((NC, NPAD, FH), jnp.float32),
            jax.ShapeDtypeStruct((NC, NPAD, FH), jnp.float32),
        ),
        mesh=mesh,
        compiler_params=pltpu.CompilerParams(use_tc_tiling_on_sc=False),
        scratch_types=[
            pltpu.VMEM((NBUF, CHUNK), jnp.int32),
            pltpu.VMEM((NBUF, CHUNK), jnp.int32),
            pltpu.VMEM((NBUF * vpc, LANES), jnp.float32),
            pltpu.VMEM((NBUF, CHUNK, FH), jnp.float32),
            pltpu.VMEM_SHARED((NPAD, FH), jnp.float32),
            pltpu.VMEM_SHARED((NPAD, FH), jnp.float32),
            [pltpu.SemaphoreType.DMA] * NBUF,
            [pltpu.SemaphoreType.DMA] * NBUF,
            [pltpu.SemaphoreType.DMA] * NBUF,
            pltpu.SemaphoreType.DMA,
            pltpu.SemaphoreType.DMA,
        ],
    )
    def cheb(y_hbm, cols_hbm, rows_hbm, vals_hbm, t1_hbm, s2_hbm,
             cidx, ridx, vvals, g, spa, spb, semi, semg, sems, semwb, semz):
        c = lax.axis_index("c")
        s = lax.axis_index("s")
        base = s * rpt

        def issue_idx(b, chunk):
            pltpu.async_copy(cols_hbm.at[s, chunk], cidx.at[b], semi[b])
            pltpu.async_copy(rows_hbm.at[s, chunk], ridx.at[b], semi[b])
            pltpu.async_copy(
                vals_hbm.at[s, pl.ds(chunk * vpc, vpc)],
                vvals.at[pl.ds(b * vpc, vpc)], semi[b],
            )

        def wait_idx(b):
            pltpu.make_async_copy(cols_hbm.at[s, 0], cidx.at[b], semi[b]).wait()
            pltpu.make_async_copy(rows_hbm.at[s, 0], ridx.at[b], semi[b]).wait()
            pltpu.make_async_copy(
                vals_hbm.at[s, pl.ds(0, vpc)],
                vvals.at[pl.ds(b * vpc, vpc)], semi[b],
            ).wait()

        def zero_acc(dst):
            def zero_g(i, carry):
                for f in range(FH // LANES):
                    g[0, i, pl.ds(f * LANES, LANES)] = jnp.zeros(
                        (LANES,), jnp.float32
                    )
                return carry

            lax.fori_loop(0, CHUNK, zero_g, 0)
            for off in range(0, rpt, CHUNK):
                pltpu.async_copy(g.at[0], dst.at[pl.ds(base + off, CHUNK)], semz)

        def drain_zero(dst):
            for off in range(0, rpt, CHUNK):
                pltpu.make_async_copy(
                    g.at[0], dst.at[pl.ds(base + off, CHUNK)], semz
                ).wait()

        def run_pass(src, dst):
            def issue_gather(b):
                pltpu.async_copy(src.at[cidx.at[b]], g.at[b], semg[b])

            def wait_gather(b):
                pltpu.make_async_copy(
                    src.at[pl.ds(0, CHUNK)], g.at[b], semg[b]
                ).wait()

            def issue_scatter(b):
                pltpu.async_copy(g.at[b], dst.at[ridx.at[b]], sems[b], add=True)

            def wait_scatter(b):
                pltpu.make_async_copy(
                    g.at[b], dst.at[pl.ds(0, CHUNK)], sems[b]
                ).wait()

            for b in range(3):
                wait_idx(b)
                issue_gather(b)

            def round_body(k, carry):
                for b in range(NBUF):
                    wait_gather(b)

                    @plsc.parallel_loop(0, vpc, 1, unroll=4)
                    def scale_group(t):
                        vv16 = vvals[b * vpc + t, :]
                        for l in range(LANES):
                            vv = jnp.full((LANES,), vv16[l], jnp.float32)
                            row = t * LANES + l
                            for f in range(FH // LANES):
                                sl = pl.ds(f * LANES, LANES)
                                g[b, row, sl] = g[b, row, sl] * vv

                    issue_scatter(b)

                    bA = (b + 3) % NBUF

                    def _refill_a():
                        wait_scatter(bA)
                        issue_idx(bA, k * NBUF + b + 3)

                    if b < 2:
                        @pl.when(k > 0)
                        def _():
                            _refill_a()
                    else:
                        @pl.when(k < nrounds - 1)
                        def _():
                            _refill_a()

                    bB = (b + 2) % NBUF

                    def _refill_b():
                        wait_idx(bB)
                        issue_gather(bB)

                    if b == 0:
                        @pl.when(k > 0)
                        def _():
                            _refill_b()
                    elif b < 3:
                        _refill_b()
                    else:
                        @pl.when(k < nrounds - 1)
                        def _():
                            _refill_b()
                return carry

            lax.fori_loop(0, nrounds, round_body, 0)
            for b in range(NBUF):
                wait_scatter(b)

        for b in range(NBUF):
            issue_idx(b, b)
        zero_acc(spb)

        n_y = y_hbm.shape[0]
        for cc in range(NC):
            stage_rows = min(rpt, n_y - (NS - 1) * rpt)

            @pl.when((c == cc) & (s < NS - 1))
            def _():
                pltpu.sync_copy(
                    y_hbm.at[pl.ds(base, rpt), pl.ds(cc * FH, FH)],
                    spa.at[pl.ds(base, rpt)],
                )

            @pl.when((c == cc) & (s == NS - 1))
            def _():
                pltpu.sync_copy(
                    y_hbm.at[pl.ds((NS - 1) * rpt, stage_rows), pl.ds(cc * FH, FH)],
                    spa.at[pl.ds((NS - 1) * rpt, stage_rows)],
                )

        drain_zero(spb)
        plsc.subcore_barrier()

        run_pass(spa, spb)
        plsc.subcore_barrier()

        wb = pltpu.async_copy(
            spb.at[pl.ds(base, rpt)], t1_hbm.at[c, pl.ds(base, rpt)], semwb
        )
        for b in range(NBUF):
            issue_idx(b, b)
        zero_acc(spa)
        drain_zero(spa)
        plsc.subcore_barrier()

        run_pass(spb, spa)
        plsc.subcore_barrier()
        pltpu.sync_copy(spa.at[pl.ds(base, rpt)], s2_hbm.at[c, pl.ds(base, rpt)])
        wb.wait()

    return cheb


_BR = 2000


def _finalize(x, t1, s2, theta, bias2d):
    def body(x_ref, t1_ref, s2_ref, th_ref, bias_ref, o_ref):
        wa = th_ref[0] - th_ref[2]
        wb = th_ref[1]
        wc = 2.0 * th_ref[2]
        o = (
            jnp.dot(x_ref[...], wa, preferred_element_type=jnp.float32)
            + jnp.dot(t1_ref[0], wb[:FH], preferred_element_type=jnp.float32)
            + jnp.dot(t1_ref[1], wb[FH:], preferred_element_type=jnp.float32)
            + jnp.dot(s2_ref[0], wc[:FH], preferred_element_type=jnp.float32)
            + jnp.dot(s2_ref[1], wc[FH:], preferred_element_type=jnp.float32)
        )
        o_ref[...] = jnp.maximum(o + bias_ref[...], 0.0)

    return pl.pallas_call(
        body,
        grid=(N // _BR,),
        in_specs=[
            pl.BlockSpec((_BR, F), lambda i: (i, 0)),
            pl.BlockSpec((NC, _BR, FH), lambda i: (0, i, 0)),
            pl.BlockSpec((NC, _BR, FH), lambda i: (0, i, 0)),
            pl.BlockSpec((3, F, F), lambda i: (0, 0, 0)),
            pl.BlockSpec((1, F), lambda i: (0, 0)),
        ],
        out_specs=pl.BlockSpec((_BR, F), lambda i: (i, 0)),
        out_shape=jax.ShapeDtypeStruct((N, F), jnp.float32),
    )(x, t1, s2, theta, bias2d)


def kernel(x, rows, cols, vals, theta, bias):
    x = x.astype(jnp.float32)
    e = rows.shape[0]
    per_super = NS * CHUNK
    nchunks = -(-e // per_super)
    nchunks = -(-nchunks // NBUF) * NBUF
    e_pad = nchunks * per_super
    pad = e_pad - e
    if pad:
        spread = (jnp.arange(pad, dtype=jnp.int32) * 37) % N
        rows = jnp.concatenate([rows, spread])
        cols = jnp.concatenate([cols, spread])
        vals = jnp.pad(vals, (0, pad))
    cols2 = cols.reshape(NS, nchunks, 128)
    rows2 = rows.reshape(NS, nchunks, 128)
    vals2 = vals.reshape(NS, nchunks * (CHUNK // LANES), LANES)

    t1, s2 = _cheb_sc(nchunks)(x, cols2, rows2, vals2)
    return _finalize(x, t1, s2, theta, bias.reshape(1, F))
